# Initial kernel scaffold; baseline (speedup 1.0000x reference)
import functools

import jax
import jax.numpy as jnp
from jax import lax
from jax.experimental import pallas as pl
from jax.experimental.pallas import tpu as pltpu
from jax.experimental.pallas import tpu_sc as plsc

NUM_CELLS = 100000
EMBED_DIM = 64
BATCH = 16384

NUM_CORES = 2
NUM_SUBCORES = 16
NUM_WORKERS = NUM_CORES * NUM_SUBCORES
B_PER_W = BATCH // NUM_WORKERS
CHUNK = 128
NUM_CHUNKS = B_PER_W // CHUNK

_mesh = plsc.VectorSubcoreMesh(core_axis_name="c", subcore_axis_name="s")


@functools.partial(
    pl.kernel,
    mesh=_mesh,
    out_type=jax.ShapeDtypeStruct((BATCH, EMBED_DIM), jnp.float32),
    scratch_types=[
        pltpu.VMEM((NUM_CHUNKS, CHUNK), jnp.int32),
        pltpu.VMEM((B_PER_W, EMBED_DIM), jnp.float32),
        pltpu.SemaphoreType.DMA,
    ],
)
def _sc_gather(idx_hbm, table_hbm, out_hbm, idx_v, rows_v, sem):
    wid = lax.axis_index("s") * NUM_CORES + lax.axis_index("c")
    base = wid * B_PER_W
    pltpu.sync_copy(idx_hbm.at[wid], idx_v)
    copies = [
        pltpu.async_copy(
            table_hbm.at[idx_v.at[j]],
            rows_v.at[pl.ds(j * CHUNK, CHUNK)],
            sem,
        )
        for j in range(NUM_CHUNKS)
    ]
    for c in copies:
        c.wait()
    pltpu.sync_copy(rows_v, out_hbm.at[pl.ds(base, B_PER_W)])


def kernel(h3_ids, table):
    idx = h3_ids.astype(jnp.int32).reshape(NUM_WORKERS, NUM_CHUNKS, CHUNK)
    return _sc_gather(idx, table)


# speedup vs baseline: 1.1955x; 1.1955x over previous
import functools

import jax
import jax.numpy as jnp
from jax import lax
from jax.experimental import pallas as pl
from jax.experimental.pallas import tpu as pltpu
from jax.experimental.pallas import tpu_sc as plsc

NUM_CELLS = 100000
EMBED_DIM = 64
BATCH = 16384

NUM_CORES = 2
NUM_SUBCORES = 16
NUM_WORKERS = NUM_CORES * NUM_SUBCORES
ROWS_PER_W = EMBED_DIM // NUM_WORKERS
CHUNK_B = 4096
N_CHUNKS_B = BATCH // CHUNK_B
LANES = 16

_mesh = plsc.VectorSubcoreMesh(core_axis_name="c", subcore_axis_name="s")


@functools.partial(
    pl.kernel,
    mesh=_mesh,
    out_type=jax.ShapeDtypeStruct((EMBED_DIM, BATCH), jnp.float32),
    scratch_types=[
        pltpu.VMEM((NUM_CELLS,), jnp.float32),
        pltpu.VMEM((CHUNK_B,), jnp.int32),
        pltpu.VMEM((CHUNK_B,), jnp.float32),
    ],
    compiler_params=pltpu.CompilerParams(
        use_tc_tiling_on_sc=True, needs_layout_passes=False
    ),
)
def _sc_gather_t(idx_hbm, tbl_t_hbm, out_t_hbm, row_v, idx_v, out_v):
    wid = lax.axis_index("s") * NUM_CORES + lax.axis_index("c")

    def gather_chunk(k, _):
        iv = idx_v[pl.ds(k * LANES, LANES)]
        out_v[pl.ds(k * LANES, LANES)] = plsc.load_gather(row_v, [iv])
        return _

    for r in range(ROWS_PER_W):
        d = wid + NUM_WORKERS * r
        pltpu.sync_copy(tbl_t_hbm.at[d], row_v)
        for c in range(N_CHUNKS_B):
            pltpu.sync_copy(idx_hbm.at[pl.ds(c * CHUNK_B, CHUNK_B)], idx_v)
            lax.fori_loop(0, CHUNK_B // LANES, gather_chunk, 0, unroll=4)
            pltpu.sync_copy(out_v, out_t_hbm.at[d, pl.ds(c * CHUNK_B, CHUNK_B)])


def kernel(h3_ids, table):
    out_t = _sc_gather_t(h3_ids.astype(jnp.int32), table.T)
    return out_t.T


# speedup vs baseline: 1.9805x; 1.6566x over previous
import functools

import jax
import jax.numpy as jnp
from jax import lax
from jax.experimental import pallas as pl
from jax.experimental.pallas import tpu as pltpu
from jax.experimental.pallas import tpu_sc as plsc

NUM_CELLS = 100000
EMBED_DIM = 64
BATCH = 16384

NUM_CORES = 2
NUM_SUBCORES = 16
NUM_WORKERS = NUM_CORES * NUM_SUBCORES
ROWS_PER_W = EMBED_DIM // NUM_WORKERS
CHUNK_B = 4096
N_CHUNKS_B = BATCH // CHUNK_B
LANES = 16

_mesh = plsc.VectorSubcoreMesh(core_axis_name="c", subcore_axis_name="s")


@functools.partial(
    pl.kernel,
    mesh=_mesh,
    out_type=jax.ShapeDtypeStruct((EMBED_DIM, BATCH), jnp.float32),
    scratch_types=[
        pltpu.VMEM((NUM_CELLS,), jnp.float32),
        pltpu.VMEM((BATCH,), jnp.int32),
        pltpu.VMEM((CHUNK_B,), jnp.float32),
        pltpu.VMEM((CHUNK_B,), jnp.float32),
        pltpu.SemaphoreType.DMA,
        pltpu.SemaphoreType.DMA,
        pltpu.SemaphoreType.DMA,
    ],
    compiler_params=pltpu.CompilerParams(
        use_tc_tiling_on_sc=True, needs_layout_passes=False
    ),
)
def _sc_gather_t(idx_hbm, tbl_t_hbm, out_t_hbm,
                 row_v, idx_v, out_a, out_b, sem_row, sem_a, sem_b):
    wid = lax.axis_index("s") * NUM_CORES + lax.axis_index("c")
    bufs = (out_a, out_b)
    sems = (sem_a, sem_b)

    row_cp = pltpu.async_copy(tbl_t_hbm.at[wid], row_v, sem_row)
    pltpu.sync_copy(idx_hbm, idx_v)
    row_cp.wait()

    pending = [None, None]
    for r in range(ROWS_PER_W):
        d = wid + NUM_WORKERS * r
        for c in range(N_CHUNKS_B):
            slot = c % 2
            buf, sem = bufs[slot], sems[slot]
            if pending[slot] is not None:
                pending[slot].wait()
                pending[slot] = None

            def gather_chunk(k, _buf=buf, _c=c):
                iv = idx_v[pl.ds(_c * CHUNK_B + k, LANES)]
                _buf[pl.ds(k, LANES)] = plsc.load_gather(row_v, [iv])

            plsc.parallel_loop(0, CHUNK_B, step=LANES, unroll=8)(gather_chunk)

            if r + 1 < ROWS_PER_W and c == N_CHUNKS_B - 1:
                row_cp = pltpu.async_copy(
                    tbl_t_hbm.at[wid + NUM_WORKERS * (r + 1)], row_v, sem_row
                )
            pending[slot] = pltpu.async_copy(
                buf, out_t_hbm.at[d, pl.ds(c * CHUNK_B, CHUNK_B)], sem
            )
        if r + 1 < ROWS_PER_W:
            row_cp.wait()
    for p in pending:
        if p is not None:
            p.wait()


def kernel(h3_ids, table):
    out_t = _sc_gather_t(h3_ids.astype(jnp.int32), table.T)
    return out_t.T
